# speedup vs baseline: 1.6264x; 1.6264x over previous
import functools

import jax
import jax.numpy as jnp
from jax import lax
from jax.experimental import pallas as pl
from jax.experimental.pallas import tpu as pltpu
from jax.experimental.pallas import tpu_sc as plsc

NUM_CODEBOOKS = 32
CODEBOOK_VOCAB_SIZE = 2051
HIDDEN = 2048
BATCH = 4
SEQ = 2048

_INFO = plsc.get_sparse_core_info()
_NC = _INFO.num_cores
_NS = _INFO.num_subcores
NW = _NC * _NS
TOTAL = BATCH * SEQ
BPW = TOTAL // NW
CHUNK = 16
NCHUNK = BPW // CHUNK


def _body(ids_hbm, cb_hbm, table_hbm, out_hbm, idx_v, cb_v, rows_v, sem0, sem1):
    wid = lax.axis_index("s") * _NC + lax.axis_index("c")
    base = wid * BPW

    pltpu.sync_copy(ids_hbm.at[wid], idx_v)
    pltpu.sync_copy(cb_hbm.at[wid], cb_v)

    for c in range(NCHUNK):
        idx_v[c, :] = idx_v[c, :] + cb_v[c, :] * CODEBOOK_VOCAB_SIZE

    sems = [sem0, sem1]
    cps = [None, None]

    def start(c):
        cps[c % 2] = pltpu.async_copy(
            table_hbm.at[idx_v.at[c]], rows_v.at[c % 2], sems[c % 2]
        )

    start(0)
    for c in range(NCHUNK):
        if c + 1 < NCHUNK:
            start(c + 1)
        cps[c % 2].wait()
        pltpu.sync_copy(rows_v.at[c % 2], out_hbm.at[pl.ds(base + c * CHUNK, CHUNK)])


@jax.jit
def kernel(input_ids, codebook_idxs, embed_audio_tokens_weight):
    ids3 = input_ids.astype(jnp.int32).reshape(NW, NCHUNK, CHUNK)
    cb3 = codebook_idxs.astype(jnp.int32).reshape(NW, NCHUNK, CHUNK)

    mesh = plsc.VectorSubcoreMesh(core_axis_name="c", subcore_axis_name="s")
    run = functools.partial(
        pl.kernel,
        mesh=mesh,
        out_type=jax.ShapeDtypeStruct((TOTAL, HIDDEN), jnp.float32),
        scratch_types=[
            pltpu.VMEM((NCHUNK, CHUNK), jnp.int32),
            pltpu.VMEM((NCHUNK, CHUNK), jnp.int32),
            pltpu.VMEM((2, CHUNK, HIDDEN), jnp.float32),
            pltpu.SemaphoreType.DMA,
            pltpu.SemaphoreType.DMA,
        ],
    )(_body)
    out = run(ids3, cb3, embed_audio_tokens_weight)
    return out.reshape(BATCH, SEQ, HIDDEN)


# speedup vs baseline: 1.6474x; 1.0129x over previous
import functools

import jax
import jax.numpy as jnp
from jax import lax
from jax.experimental import pallas as pl
from jax.experimental.pallas import tpu as pltpu
from jax.experimental.pallas import tpu_sc as plsc

NUM_CODEBOOKS = 32
CODEBOOK_VOCAB_SIZE = 2051
HIDDEN = 2048
BATCH = 4
SEQ = 2048

_INFO = plsc.get_sparse_core_info()
_NC = _INFO.num_cores
_NS = _INFO.num_subcores
NW = _NC * _NS
TOTAL = BATCH * SEQ
BPW = TOTAL // NW
CHUNK = 16
NCHUNK = BPW // CHUNK


NBUF = 3


def _body(ids_hbm, cb_hbm, table_hbm, out_hbm, idx_v, cb_v, rows_v, *sems):
    wid = lax.axis_index("s") * _NC + lax.axis_index("c")
    base = wid * BPW

    pltpu.sync_copy(ids_hbm.at[wid], idx_v)
    pltpu.sync_copy(cb_hbm.at[wid], cb_v)

    for c in range(NCHUNK):
        idx_v[c, :] = idx_v[c, :] + cb_v[c, :] * CODEBOOK_VOCAB_SIZE

    g_sems, o_sems = sems[:NBUF], sems[NBUF:]
    g_cps = [None] * NBUF
    o_cps = [None] * NBUF

    def fire_gather(c):
        b = c % NBUF
        g_cps[b] = pltpu.async_copy(
            table_hbm.at[idx_v.at[c]], rows_v.at[b], g_sems[b]
        )

    def fire_out(c):
        b = c % NBUF
        o_cps[b] = pltpu.async_copy(
            rows_v.at[b], out_hbm.at[pl.ds(base + c * CHUNK, CHUNK)], o_sems[b]
        )

    for c in range(NBUF):
        fire_gather(c)
    for c in range(NCHUNK):
        b = c % NBUF
        g_cps[b].wait()
        fire_out(c)
        if c + NBUF < NCHUNK:
            o_cps[b].wait()
            fire_gather(c + NBUF)
    for c in range(NCHUNK - NBUF, NCHUNK):
        o_cps[c % NBUF].wait()


@jax.jit
def kernel(input_ids, codebook_idxs, embed_audio_tokens_weight):
    ids3 = input_ids.astype(jnp.int32).reshape(NW, NCHUNK, CHUNK)
    cb3 = codebook_idxs.astype(jnp.int32).reshape(NW, NCHUNK, CHUNK)

    mesh = plsc.VectorSubcoreMesh(core_axis_name="c", subcore_axis_name="s")
    run = functools.partial(
        pl.kernel,
        mesh=mesh,
        out_type=jax.ShapeDtypeStruct((TOTAL, HIDDEN), jnp.float32),
        scratch_types=[
            pltpu.VMEM((NCHUNK, CHUNK), jnp.int32),
            pltpu.VMEM((NCHUNK, CHUNK), jnp.int32),
            pltpu.VMEM((NBUF, CHUNK, HIDDEN), jnp.float32),
        ] + [pltpu.SemaphoreType.DMA] * (2 * NBUF),
    )(_body)
    out = run(ids3, cb3, embed_audio_tokens_weight)
    return out.reshape(BATCH, SEQ, HIDDEN)


# speedup vs baseline: 1.6574x; 1.0061x over previous
import functools

import jax
import jax.numpy as jnp
from jax import lax
from jax.experimental import pallas as pl
from jax.experimental.pallas import tpu as pltpu
from jax.experimental.pallas import tpu_sc as plsc

NUM_CODEBOOKS = 32
CODEBOOK_VOCAB_SIZE = 2051
HIDDEN = 2048
BATCH = 4
SEQ = 2048

_INFO = plsc.get_sparse_core_info()
_NC = _INFO.num_cores
_NS = _INFO.num_subcores
NW = _NC * _NS
TOTAL = BATCH * SEQ
BPW = TOTAL // NW
CHUNK = 16
NCHUNK = BPW // CHUNK


NBUF = 3


def _body(idscb_hbm, table_hbm, out_hbm, stg_v, idx_v, rows_v, *sems):
    wid = lax.axis_index("s") * _NC + lax.axis_index("c")
    base = wid * BPW

    pltpu.sync_copy(idscb_hbm.at[wid], stg_v)

    for c in range(NCHUNK):
        idx_v[c, :] = stg_v[0, c, :] + stg_v[1, c, :] * CODEBOOK_VOCAB_SIZE

    g_sems, o_sems = sems[:NBUF], sems[NBUF:]
    g_cps = [None] * NBUF
    o_cps = [None] * NBUF

    def fire_gather(c):
        b = c % NBUF
        g_cps[b] = pltpu.async_copy(
            table_hbm.at[idx_v.at[c]], rows_v.at[b], g_sems[b]
        )

    def fire_out(c):
        b = c % NBUF
        o_cps[b] = pltpu.async_copy(
            rows_v.at[b], out_hbm.at[pl.ds(base + c * CHUNK, CHUNK)], o_sems[b]
        )

    for c in range(NBUF):
        fire_gather(c)
    for c in range(NCHUNK):
        b = c % NBUF
        g_cps[b].wait()
        fire_out(c)
        if c + NBUF < NCHUNK:
            o_cps[b].wait()
            fire_gather(c + NBUF)
    for c in range(NCHUNK - NBUF, NCHUNK):
        o_cps[c % NBUF].wait()


@jax.jit
def kernel(input_ids, codebook_idxs, embed_audio_tokens_weight):
    ids3 = input_ids.astype(jnp.int32).reshape(NW, 1, NCHUNK, CHUNK)
    cb3 = codebook_idxs.astype(jnp.int32).reshape(NW, 1, NCHUNK, CHUNK)
    idscb = jnp.concatenate([ids3, cb3], axis=1)

    mesh = plsc.VectorSubcoreMesh(core_axis_name="c", subcore_axis_name="s")
    run = functools.partial(
        pl.kernel,
        mesh=mesh,
        out_type=jax.ShapeDtypeStruct((TOTAL, HIDDEN), jnp.float32),
        scratch_types=[
            pltpu.VMEM((2, NCHUNK, CHUNK), jnp.int32),
            pltpu.VMEM((NCHUNK, CHUNK), jnp.int32),
            pltpu.VMEM((NBUF, CHUNK, HIDDEN), jnp.float32),
        ] + [pltpu.SemaphoreType.DMA] * (2 * NBUF),
    )(_body)
    out = run(idscb, embed_audio_tokens_weight)
    return out.reshape(BATCH, SEQ, HIDDEN)
